# baseline (speedup 1.0000x reference)
import functools

import jax
import jax.numpy as jnp
from jax import lax
from jax.experimental import pallas as pl
from jax.experimental.pallas import tpu as pltpu
from jax.experimental.pallas import tpu_sc as plsc

B = 4096
H = 32
HIST = 50
NW = 32
EPW = B // NW
G = 16
NG = EPW // G
ROWS_G = G * HIST
_F32 = jnp.float32
_I32 = jnp.int32


def _collapse_body(au_w2, au_b1, au_b2, au_w1a, au_w1x, au_w1b,
                   lw1q, lw1h, lw1r, lb1, lw2, lb2, lw3, lb3, rtab,
                   out_ref):
    hi = jax.lax.Precision.HIGHEST
    dot = functools.partial(jnp.dot, precision=hi)
    v2 = 0.5 * au_w2[...]
    c_h = dot(v2, au_w1a[...])
    c64 = dot(v2, au_w1x[...])[0, 0]
    c_q = dot(v2, au_w1b[...])
    dd = dot(v2, au_b1[...])[0, 0] + au_b2[0, 0]
    w32 = 0.25 * dot(lw3[...], lw2[...])
    m_q = dot(w32, lw1q[...])
    m_h = dot(w32, lw1h[...])
    m_r = dot(w32, lw1r[...])
    c0 = (dot(w32, lb1[...])[0, 0]
          + 0.5 * dot(lw3[...], lb2[...])[0, 0] + lb3[0, 0])
    rho = lax.dot_general(m_r, rtab[...], (((1,), (1,)), ((), ())),
                          precision=hi)
    row0 = jnp.concatenate([m_q, c_q], axis=1)
    row1 = jnp.concatenate([c_h, m_h], axis=1)
    row2 = jnp.concatenate([rho, jnp.zeros((1, 118), _F32)], axis=1)
    i = lax.broadcasted_iota(_I32, (1, 128), 1)
    row3 = (jnp.where(i == 0, c64, 0.0) + jnp.where(i == 1, dd, 0.0)
            + jnp.where(i == 2, c0, 0.0))
    out_ref[...] = jnp.concatenate([row0, row1, row2, row3], axis=0)


def _collapse(d, interpret=False):
    return pl.pallas_call(
        _collapse_body,
        out_shape=jax.ShapeDtypeStruct((4, 128), _F32),
        interpret=interpret,
    )(*d)


def _sc_body(itab, ctab, iid_h, aid_h, hiid_h, haid_h, hrate_h, pv_h,
             z_h,
             iid_v, aid_v, hiid_v, haid_v, hrate_v, pv_v,
             qitem, qcate, hitem, hcate, Abuf, Nbuf, simbuf,
             vbuf, ibuf, zbuf, sem_i, sem_c):
    cid = lax.axis_index("c")
    sid = lax.axis_index("s")
    wid = sid * 2 + cid
    base = pl.multiple_of(wid * EPW, EPW)
    hbase = pl.multiple_of(wid * (EPW * HIST), EPW * HIST)

    pltpu.sync_copy(pv_h, pv_v)
    pltpu.sync_copy(iid_h.at[pl.ds(base, EPW)], iid_v)
    pltpu.sync_copy(aid_h.at[pl.ds(base, EPW)], aid_v)
    pltpu.sync_copy(hiid_h.at[pl.ds(hbase, EPW * HIST)], hiid_v)
    pltpu.sync_copy(haid_h.at[pl.ds(hbase, EPW * HIST)], haid_v)
    pltpu.sync_copy(hrate_h.at[pl.ds(hbase, EPW * HIST)], hrate_v)
    pltpu.async_copy(itab.at[iid_v], qitem, sem_i).wait()
    pltpu.async_copy(ctab.at[aid_v], qcate, sem_c).wait()

    lane = lax.iota(_I32, 16)
    lane50 = lane * HIST
    neg = jnp.full((16,), -3.0e38, _F32)
    chunks = (128, 128, 128, 128, 128, 128, 32)

    def group_body(g, _):
        off = pl.multiple_of(g * ROWS_G, ROWS_G)
        item_copies, cate_copies = [], []
        pos = 0
        for ch in chunks:
            item_copies.append(pltpu.async_copy(
                itab.at[hiid_v.at[pl.ds(off + pos, ch)]],
                hitem.at[pl.ds(pos, ch)], sem_i))
            pos += ch
        pos = 0
        for ch in chunks:
            cate_copies.append(pltpu.async_copy(
                ctab.at[haid_v.at[pl.ds(off + pos, ch)]],
                hcate.at[pl.ds(pos, ch)], sem_c))
            pos += ch

        row3c = pv_v[3, pl.ds(0, 16)]
        c64s = row3c[0]
        dds = row3c[1]
        c0s = row3c[2]

        ge = g * 16 + lane
        qm = jnp.zeros((16,), _F32)
        qc = jnp.zeros((16,), _F32)
        nq2 = jnp.zeros((16,), _F32)
        r0 = jnp.zeros((16,), _I32)
        qiv = []
        for f in range(32):
            fx = jnp.bitwise_xor(lane, f)
            v = plsc.load_gather(qitem, [ge, fx])
            qiv.append(v)
            qm = qm + v * plsc.load_gather(pv_v, [r0, fx])
            qc = qc + v * plsc.load_gather(pv_v, [r0, 64 + fx])
            nq2 = nq2 + v * v

        for cp in item_copies:
            cp.wait()

        def t1_body(t, _):
            row = lane50 + t
            A = jnp.zeros((16,), _F32)
            n2 = jnp.zeros((16,), _F32)
            for f in range(32):
                fx = jnp.bitwise_xor(lane, f)
                a = plsc.load_gather(hitem, [row, fx])
                A = A + a * qiv[f]
                n2 = n2 + a * a
            Abuf[t] = A
            Nbuf[t] = n2
            return 0

        lax.fori_loop(0, HIST, t1_body, 0, unroll=2)

        qcv = []
        for f in range(32):
            fx = jnp.bitwise_xor(lane, f)
            v = plsc.load_gather(qcate, [ge, fx])
            qcv.append(v)
            qm = qm + v * plsc.load_gather(pv_v, [r0, 32 + fx])
            qc = qc + v * plsc.load_gather(pv_v, [r0, 96 + fx])
            nq2 = nq2 + v * v
        qc = qc + dds

        for cp in cate_copies:
            cp.wait()

        def t2_body(t, _):
            row = lane50 + t
            A = Abuf[t]
            n2 = Nbuf[t]
            for f in range(32):
                fx = jnp.bitwise_xor(lane, f)
                b = plsc.load_gather(hcate, [row, fx])
                A = A + b * qcv[f]
                n2 = n2 + b * b
            Abuf[t] = A
            s = jnp.maximum(nq2 * n2, 1e-30)
            si = plsc.bitcast(s, _I32)
            y = plsc.bitcast(jnp.int32(0x5F3759DF) - (si >> 1), _F32)
            hs = 0.5 * s
            for _ in range(2):
                y = y * (1.5 - hs * y * y)
            simbuf[t] = A * y
            return 0

        lax.fori_loop(0, HIST, t2_body, 0, unroll=2)

        ssum = jnp.full((16,), 1e-8, _F32)
        for k in range(5):
            bv = neg
            bi = jnp.zeros((16,), _I32)
            for t in range(HIST):
                v = simbuf[t]
                better = v > bv
                bv = jnp.where(better, v, bv)
                bi = jnp.where(better, t, bi)
            plsc.store_scatter(simbuf, [bi, lane], neg)
            vbuf[k] = bv
            ibuf[k] = bi
            ssum = ssum + bv

        def k_body(k, contrib):
            bv = vbuf[k]
            bi = ibuf[k]
            wk = bv / ssum
            row = lane50 + bi
            gk = jnp.zeros((16,), _F32)
            pk = jnp.zeros((16,), _F32)
            r1 = jnp.full((16,), 1, _I32)
            for f in range(32):
                fx = jnp.bitwise_xor(lane, f)
                a = plsc.load_gather(hitem, [row, fx])
                b = plsc.load_gather(hcate, [row, fx])
                gk = (gk + a * plsc.load_gather(pv_v, [r1, fx])
                      + b * plsc.load_gather(pv_v, [r1, 32 + fx]))
                pk = (pk + a * plsc.load_gather(pv_v, [r1, 64 + fx])
                      + b * plsc.load_gather(pv_v, [r1, 96 + fx]))
            Ak = plsc.load_gather(Abuf, [bi, lane])
            rate = plsc.load_gather(hrate_v, [ge * HIST + bi])
            rk = plsc.load_gather(pv_v, [jnp.full((16,), 2, _I32), rate])
            au = wk * (gk + Ak * c64s) + qc
            return contrib + au * (wk * pk + rk)

        contrib = lax.fori_loop(0, 5, k_body, jnp.zeros((16,), _F32))
        zbuf[pl.ds(g * 16, 16)] = qm + contrib + c0s
        return 0

    lax.fori_loop(0, NG, group_body, 0, unroll=False)
    pltpu.sync_copy(zbuf, z_h.at[pl.ds(base, EPW)])


def _sc_call(itab, ctab, iid, aid, hiid, haid, hrate, pv, interpret=False):
    mesh = plsc.VectorSubcoreMesh(core_axis_name="c", subcore_axis_name="s",
                                  num_cores=2, num_subcores=16)
    f = pl.kernel(
        _sc_body,
        out_type=jax.ShapeDtypeStruct((B,), _F32),
        mesh=mesh,
        scratch_types=[
            pltpu.VMEM((EPW,), _I32),
            pltpu.VMEM((EPW,), _I32),
            pltpu.VMEM((EPW * HIST,), _I32),
            pltpu.VMEM((EPW * HIST,), _I32),
            pltpu.VMEM((EPW * HIST,), _I32),
            pltpu.VMEM((4, 128), _F32),
            pltpu.VMEM((EPW, H), _F32),
            pltpu.VMEM((EPW, H), _F32),
            pltpu.VMEM((ROWS_G, H), _F32),
            pltpu.VMEM((ROWS_G, H), _F32),
            pltpu.VMEM((HIST, 16), _F32),
            pltpu.VMEM((HIST, 16), _F32),
            pltpu.VMEM((HIST, 16), _F32),
            pltpu.VMEM((8, 16), _F32),
            pltpu.VMEM((8, 16), _I32),
            pltpu.VMEM((EPW,), _F32),
            pltpu.SemaphoreType.DMA,
            pltpu.SemaphoreType.DMA,
        ],
        compiler_params=pltpu.CompilerParams(needs_layout_passes=False,
                                             use_tc_tiling_on_sc=False),
        interpret=interpret,
    )
    return f(itab, ctab, iid, aid, hiid, haid, hrate, pv)


def _final_body(z_ref, y_ref, probs_ref, loss_ref):
    z = z_ref[...]
    y = y_ref[...]
    probs_ref[...] = jax.nn.sigmoid(z)
    l = jnp.maximum(z, 0.0) - z * y + jnp.log1p(jnp.exp(-jnp.abs(z)))
    loss_ref[0, 0] = jnp.sum(l) * (1.0 / B)


def _final(z2d, y2d, interpret=False):
    return pl.pallas_call(
        _final_body,
        out_shape=(jax.ShapeDtypeStruct((B // 128, 128), _F32),
                   jax.ShapeDtypeStruct((1, 1), _F32)),
        out_specs=(pl.BlockSpec((B // 128, 128), lambda: (0, 0)),
                   pl.BlockSpec(memory_space=pltpu.SMEM)),
        interpret=interpret,
    )(z2d, y2d)


def _run(iid, aid, lb, hist_iid_seq, hist_aid_seq, hist_rate_seq,
         item_table, cate_table, rating_table,
         au_w1, au_b1, au_w2, au_b2,
         lin_w1, lin_b1, lin_w2, lin_b2, lin_w3, lin_b3,
         interpret=False):
    pv = _collapse((au_w2, au_b1.reshape(36, 1), au_b2.reshape(1, 1),
                    au_w1[:, 0:64], au_w1[:, 64:65], au_w1[:, 65:129],
                    lin_w1[:, 0:64], lin_w1[:, 64:128], lin_w1[:, 128:160],
                    lin_b1.reshape(80, 1), lin_w2, lin_b2.reshape(40, 1),
                    lin_w3, lin_b3.reshape(1, 1), rating_table),
                   interpret=interpret)
    z = _sc_call(item_table, cate_table,
                 iid.astype(_I32),
                 aid.reshape(B).astype(_I32),
                 hist_iid_seq.reshape(B * HIST).astype(_I32),
                 hist_aid_seq.reshape(B * HIST).astype(_I32),
                 hist_rate_seq.reshape(B * HIST).astype(_I32),
                 pv, interpret=interpret)
    y2d = lb.reshape(B // 128, 128).astype(_F32)
    probs2d, loss11 = _final(z.reshape(B // 128, 128), y2d,
                             interpret=interpret)
    return probs2d.reshape(B, 1), loss11[0, 0]


def kernel(iid, aid, lb, hist_iid_seq, hist_aid_seq, hist_rate_seq,
           item_table, cate_table, rating_table,
           au_w1, au_b1, au_w2, au_b2,
           lin_w1, lin_b1, lin_w2, lin_b2, lin_w3, lin_b3):
    return _run(iid, aid, lb, hist_iid_seq, hist_aid_seq, hist_rate_seq,
                item_table, cate_table, rating_table,
                au_w1, au_b1, au_w2, au_b2,
                lin_w1, lin_b1, lin_w2, lin_b2, lin_w3, lin_b3)


# speedup vs baseline: 1.0901x; 1.0901x over previous
import functools

import jax
import jax.numpy as jnp
from jax import lax
from jax.experimental import pallas as pl
from jax.experimental.pallas import tpu as pltpu
from jax.experimental.pallas import tpu_sc as plsc

B = 4096
H = 32
HIST = 50
NW = 32
EPW = B // NW
G = 16
NG = EPW // G
ROWS_G = G * HIST
_F32 = jnp.float32
_I32 = jnp.int32


def _collapse_body(au_w2, au_b1, au_b2, au_w1a, au_w1x, au_w1b,
                   lw1q, lw1h, lw1r, lb1, lw2, lb2, lw3, lb3, rtab,
                   out_ref):
    hi = jax.lax.Precision.HIGHEST
    dot = functools.partial(jnp.dot, precision=hi)
    v2 = 0.5 * au_w2[...]
    c_h = dot(v2, au_w1a[...])
    c64 = dot(v2, au_w1x[...])[0, 0]
    c_q = dot(v2, au_w1b[...])
    dd = dot(v2, au_b1[...])[0, 0] + au_b2[0, 0]
    w32 = 0.25 * dot(lw3[...], lw2[...])
    m_q = dot(w32, lw1q[...])
    m_h = dot(w32, lw1h[...])
    m_r = dot(w32, lw1r[...])
    c0 = (dot(w32, lb1[...])[0, 0]
          + 0.5 * dot(lw3[...], lb2[...])[0, 0] + lb3[0, 0])
    rho = lax.dot_general(m_r, rtab[...], (((1,), (1,)), ((), ())),
                          precision=hi)
    row0 = jnp.concatenate([m_q, c_q], axis=1)
    row1 = jnp.concatenate([c_h, m_h], axis=1)
    row2 = jnp.concatenate([rho, jnp.zeros((1, 118), _F32)], axis=1)
    i = lax.broadcasted_iota(_I32, (1, 128), 1)
    row3 = (jnp.where(i == 0, c64, 0.0) + jnp.where(i == 1, dd, 0.0)
            + jnp.where(i == 2, c0, 0.0))
    out_ref[...] = jnp.concatenate([row0, row1, row2, row3], axis=0)


def _collapse(d, interpret=False):
    return pl.pallas_call(
        _collapse_body,
        out_shape=jax.ShapeDtypeStruct((4, 128), _F32),
        interpret=interpret,
    )(*d)


def _sc_body(itab, ctab, iid_h, aid_h, hiid_h, haid_h, hrate_h, pv_h,
             z_h,
             iid_v, aid_v, hiid_v, haid_v, hrate_v, pv_v,
             qitem, qcate, hitem, hcate, Abuf, Nbuf, simbuf,
             vbuf, ibuf, zbuf, sem_i, sem_c):
    cid = lax.axis_index("c")
    sid = lax.axis_index("s")
    wid = sid * 2 + cid
    base = pl.multiple_of(wid * EPW, EPW)
    hbase = pl.multiple_of(wid * (EPW * HIST), EPW * HIST)

    pltpu.sync_copy(pv_h, pv_v)
    pltpu.sync_copy(iid_h.at[pl.ds(base, EPW)], iid_v)
    pltpu.sync_copy(aid_h.at[pl.ds(base, EPW)], aid_v)
    pltpu.sync_copy(hiid_h.at[pl.ds(hbase, EPW * HIST)], hiid_v)
    pltpu.sync_copy(haid_h.at[pl.ds(hbase, EPW * HIST)], haid_v)
    pltpu.sync_copy(hrate_h.at[pl.ds(hbase, EPW * HIST)], hrate_v)
    pltpu.async_copy(itab.at[iid_v], qitem, sem_i).wait()
    pltpu.async_copy(ctab.at[aid_v], qcate, sem_c).wait()

    lane = lax.iota(_I32, 16)
    lane50 = lane * HIST
    neg = jnp.full((16,), -3.0e38, _F32)
    chunks = (128, 128, 128, 128, 128, 128, 32)

    def group_body(g, _):
        off = pl.multiple_of(g * ROWS_G, ROWS_G)
        item_copies, cate_copies = [], []
        pos = 0
        for ch in chunks:
            item_copies.append(pltpu.async_copy(
                itab.at[hiid_v.at[pl.ds(off + pos, ch)]],
                hitem.at[pl.ds(pos, ch)], sem_i))
            pos += ch
        pos = 0
        for ch in chunks:
            cate_copies.append(pltpu.async_copy(
                ctab.at[haid_v.at[pl.ds(off + pos, ch)]],
                hcate.at[pl.ds(pos, ch)], sem_c))
            pos += ch

        row3c = pv_v[3, pl.ds(0, 16)]
        c64s = row3c[0]
        dds = row3c[1]
        c0s = row3c[2]

        ge = g * 16 + lane
        qm = jnp.zeros((16,), _F32)
        qc = jnp.zeros((16,), _F32)
        nq2 = jnp.zeros((16,), _F32)
        r0 = jnp.zeros((16,), _I32)
        qiv = []
        for f in range(32):
            fx = jnp.bitwise_xor(lane, f)
            v = plsc.load_gather(qitem, [ge, fx])
            qiv.append(v)
            qm = qm + v * plsc.load_gather(pv_v, [r0, fx])
            qc = qc + v * plsc.load_gather(pv_v, [r0, 64 + fx])
            nq2 = nq2 + v * v

        for cp in item_copies:
            cp.wait()

        def t1_body(t, _):
            row = lane50 + t
            A = jnp.zeros((16,), _F32)
            n2 = jnp.zeros((16,), _F32)
            for f in range(32):
                fx = jnp.bitwise_xor(lane, f)
                a = plsc.load_gather(hitem, [row, fx])
                A = A + a * qiv[f]
                n2 = n2 + a * a
            Abuf[t] = A
            Nbuf[t] = n2
            return 0

        lax.fori_loop(0, HIST, t1_body, 0, unroll=1)

        qcv = []
        for f in range(32):
            fx = jnp.bitwise_xor(lane, f)
            v = plsc.load_gather(qcate, [ge, fx])
            qcv.append(v)
            qm = qm + v * plsc.load_gather(pv_v, [r0, 32 + fx])
            qc = qc + v * plsc.load_gather(pv_v, [r0, 96 + fx])
            nq2 = nq2 + v * v
        qc = qc + dds

        for cp in cate_copies:
            cp.wait()

        def t2_body(t, _):
            row = lane50 + t
            A = Abuf[t]
            n2 = Nbuf[t]
            for f in range(32):
                fx = jnp.bitwise_xor(lane, f)
                b = plsc.load_gather(hcate, [row, fx])
                A = A + b * qcv[f]
                n2 = n2 + b * b
            Abuf[t] = A
            s = jnp.maximum(nq2 * n2, 1e-30)
            si = plsc.bitcast(s, _I32)
            y = plsc.bitcast(jnp.int32(0x5F3759DF) - (si >> 1), _F32)
            hs = 0.5 * s
            for _ in range(2):
                y = y * (1.5 - hs * y * y)
            simbuf[t] = A * y
            return 0

        lax.fori_loop(0, HIST, t2_body, 0, unroll=1)

        ssum = jnp.full((16,), 1e-8, _F32)
        for k in range(5):
            bv = neg
            bi = jnp.zeros((16,), _I32)
            for t in range(HIST):
                v = simbuf[t]
                better = v > bv
                bv = jnp.where(better, v, bv)
                bi = jnp.where(better, t, bi)
            plsc.store_scatter(simbuf, [bi, lane], neg)
            vbuf[k] = bv
            ibuf[k] = bi
            ssum = ssum + bv

        def k_body(k, contrib):
            bv = vbuf[k]
            bi = ibuf[k]
            wk = bv / ssum
            row = lane50 + bi
            gk = jnp.zeros((16,), _F32)
            pk = jnp.zeros((16,), _F32)
            r1 = jnp.full((16,), 1, _I32)
            for f in range(32):
                fx = jnp.bitwise_xor(lane, f)
                a = plsc.load_gather(hitem, [row, fx])
                b = plsc.load_gather(hcate, [row, fx])
                gk = (gk + a * plsc.load_gather(pv_v, [r1, fx])
                      + b * plsc.load_gather(pv_v, [r1, 32 + fx]))
                pk = (pk + a * plsc.load_gather(pv_v, [r1, 64 + fx])
                      + b * plsc.load_gather(pv_v, [r1, 96 + fx]))
            Ak = plsc.load_gather(Abuf, [bi, lane])
            rate = plsc.load_gather(hrate_v, [ge * HIST + bi])
            rk = plsc.load_gather(pv_v, [jnp.full((16,), 2, _I32), rate])
            au = wk * (gk + Ak * c64s) + qc
            return contrib + au * (wk * pk + rk)

        contrib = lax.fori_loop(0, 5, k_body, jnp.zeros((16,), _F32))
        zbuf[pl.ds(g * 16, 16)] = qm + contrib + c0s
        return 0

    lax.fori_loop(0, NG, group_body, 0, unroll=False)
    pltpu.sync_copy(zbuf, z_h.at[pl.ds(base, EPW)])


def _sc_call(itab, ctab, iid, aid, hiid, haid, hrate, pv, interpret=False):
    mesh = plsc.VectorSubcoreMesh(core_axis_name="c", subcore_axis_name="s",
                                  num_cores=2, num_subcores=16)
    f = pl.kernel(
        _sc_body,
        out_type=jax.ShapeDtypeStruct((B,), _F32),
        mesh=mesh,
        scratch_types=[
            pltpu.VMEM((EPW,), _I32),
            pltpu.VMEM((EPW,), _I32),
            pltpu.VMEM((EPW * HIST,), _I32),
            pltpu.VMEM((EPW * HIST,), _I32),
            pltpu.VMEM((EPW * HIST,), _I32),
            pltpu.VMEM((4, 128), _F32),
            pltpu.VMEM((EPW, H), _F32),
            pltpu.VMEM((EPW, H), _F32),
            pltpu.VMEM((ROWS_G, H), _F32),
            pltpu.VMEM((ROWS_G, H), _F32),
            pltpu.VMEM((HIST, 16), _F32),
            pltpu.VMEM((HIST, 16), _F32),
            pltpu.VMEM((HIST, 16), _F32),
            pltpu.VMEM((8, 16), _F32),
            pltpu.VMEM((8, 16), _I32),
            pltpu.VMEM((EPW,), _F32),
            pltpu.SemaphoreType.DMA,
            pltpu.SemaphoreType.DMA,
        ],
        compiler_params=pltpu.CompilerParams(needs_layout_passes=False,
                                             use_tc_tiling_on_sc=False),
        interpret=interpret,
    )
    return f(itab, ctab, iid, aid, hiid, haid, hrate, pv)


def _final_body(z_ref, y_ref, probs_ref, loss_ref):
    z = z_ref[...]
    y = y_ref[...]
    probs_ref[...] = jax.nn.sigmoid(z)
    l = jnp.maximum(z, 0.0) - z * y + jnp.log1p(jnp.exp(-jnp.abs(z)))
    loss_ref[0, 0] = jnp.sum(l) * (1.0 / B)


def _final(z2d, y2d, interpret=False):
    return pl.pallas_call(
        _final_body,
        out_shape=(jax.ShapeDtypeStruct((B // 128, 128), _F32),
                   jax.ShapeDtypeStruct((1, 1), _F32)),
        out_specs=(pl.BlockSpec((B // 128, 128), lambda: (0, 0)),
                   pl.BlockSpec(memory_space=pltpu.SMEM)),
        interpret=interpret,
    )(z2d, y2d)


def _run(iid, aid, lb, hist_iid_seq, hist_aid_seq, hist_rate_seq,
         item_table, cate_table, rating_table,
         au_w1, au_b1, au_w2, au_b2,
         lin_w1, lin_b1, lin_w2, lin_b2, lin_w3, lin_b3,
         interpret=False):
    pv = _collapse((au_w2, au_b1.reshape(36, 1), au_b2.reshape(1, 1),
                    au_w1[:, 0:64], au_w1[:, 64:65], au_w1[:, 65:129],
                    lin_w1[:, 0:64], lin_w1[:, 64:128], lin_w1[:, 128:160],
                    lin_b1.reshape(80, 1), lin_w2, lin_b2.reshape(40, 1),
                    lin_w3, lin_b3.reshape(1, 1), rating_table),
                   interpret=interpret)
    z = _sc_call(item_table, cate_table,
                 iid.astype(_I32),
                 aid.reshape(B).astype(_I32),
                 hist_iid_seq.reshape(B * HIST).astype(_I32),
                 hist_aid_seq.reshape(B * HIST).astype(_I32),
                 hist_rate_seq.reshape(B * HIST).astype(_I32),
                 pv, interpret=interpret)
    y2d = lb.reshape(B // 128, 128).astype(_F32)
    probs2d, loss11 = _final(z.reshape(B // 128, 128), y2d,
                             interpret=interpret)
    return probs2d.reshape(B, 1), loss11[0, 0]


def kernel(iid, aid, lb, hist_iid_seq, hist_aid_seq, hist_rate_seq,
           item_table, cate_table, rating_table,
           au_w1, au_b1, au_w2, au_b2,
           lin_w1, lin_b1, lin_w2, lin_b2, lin_w3, lin_b3):
    return _run(iid, aid, lb, hist_iid_seq, hist_aid_seq, hist_rate_seq,
                item_table, cate_table, rating_table,
                au_w1, au_b1, au_w2, au_b2,
                lin_w1, lin_b1, lin_w2, lin_b2, lin_w3, lin_b3)


# speedup vs baseline: 1.1534x; 1.0580x over previous
import functools

import jax
import jax.numpy as jnp
from jax import lax
from jax.experimental import pallas as pl
from jax.experimental.pallas import tpu as pltpu
from jax.experimental.pallas import tpu_sc as plsc

B = 4096
H = 32
HIST = 50
NW = 32
EPW = B // NW
G = 16
NG = EPW // G
ROWS_G = G * HIST
_F32 = jnp.float32
_I32 = jnp.int32


def _collapse_body(au_w2, au_b1, au_b2, au_w1a, au_w1x, au_w1b,
                   lw1q, lw1h, lw1r, lb1, lw2, lb2, lw3, lb3, rtab,
                   out_ref):
    hi = jax.lax.Precision.HIGHEST
    dot = functools.partial(jnp.dot, precision=hi)
    v2 = 0.5 * au_w2[...]
    c_h = dot(v2, au_w1a[...])
    c64 = dot(v2, au_w1x[...])[0, 0]
    c_q = dot(v2, au_w1b[...])
    dd = dot(v2, au_b1[...])[0, 0] + au_b2[0, 0]
    w32 = 0.25 * dot(lw3[...], lw2[...])
    m_q = dot(w32, lw1q[...])
    m_h = dot(w32, lw1h[...])
    m_r = dot(w32, lw1r[...])
    c0 = (dot(w32, lb1[...])[0, 0]
          + 0.5 * dot(lw3[...], lb2[...])[0, 0] + lb3[0, 0])
    rho = lax.dot_general(m_r, rtab[...], (((1,), (1,)), ((), ())),
                          precision=hi)
    row0 = jnp.concatenate([m_q, c_q], axis=1)
    row1 = jnp.concatenate([c_h, m_h], axis=1)
    row2 = jnp.concatenate([rho, jnp.zeros((1, 118), _F32)], axis=1)
    i = lax.broadcasted_iota(_I32, (1, 128), 1)
    row3 = (jnp.where(i == 0, c64, 0.0) + jnp.where(i == 1, dd, 0.0)
            + jnp.where(i == 2, c0, 0.0))
    out_ref[...] = jnp.concatenate([row0, row1, row2, row3], axis=0)


def _collapse(d, interpret=False):
    return pl.pallas_call(
        _collapse_body,
        out_shape=jax.ShapeDtypeStruct((4, 128), _F32),
        interpret=interpret,
    )(*d)


def _sc_body(itab, ctab, iid_h, aid_h, hiid_h, haid_h, hrate_h, pv_h,
             z_h,
             iid_v, aid_v, hiid_v, haid_v, hrate_v, pv_v,
             qitem, qcate, hitem, hcate, Abuf, Nbuf, simbuf,
             vbuf, ibuf, zbuf, sem_i0, sem_c0, sem_i1, sem_c1):
    cid = lax.axis_index("c")
    sid = lax.axis_index("s")
    wid = sid * 2 + cid
    base = pl.multiple_of(wid * EPW, EPW)
    hbase = pl.multiple_of(wid * (EPW * HIST), EPW * HIST)

    pltpu.sync_copy(pv_h, pv_v)
    pltpu.sync_copy(iid_h.at[pl.ds(base, EPW)], iid_v)
    pltpu.sync_copy(aid_h.at[pl.ds(base, EPW)], aid_v)
    pltpu.sync_copy(hiid_h.at[pl.ds(hbase, EPW * HIST)], hiid_v)
    pltpu.sync_copy(haid_h.at[pl.ds(hbase, EPW * HIST)], haid_v)
    pltpu.sync_copy(hrate_h.at[pl.ds(hbase, EPW * HIST)], hrate_v)
    pltpu.async_copy(itab.at[iid_v], qitem, sem_i0).wait()
    pltpu.async_copy(ctab.at[aid_v], qcate, sem_c0).wait()

    lane = lax.iota(_I32, 16)
    lane50 = lane * HIST
    neg = jnp.full((16,), -3.0e38, _F32)
    chunks = ((0, 128), (128, 128), (256, 128), (384, 128),
              (512, 128), (640, 128), (768, 32))

    def fire_item(g, sb, sem_i):
        off = pl.multiple_of(g * ROWS_G, ROWS_G)
        for pos, ch in chunks:
            pltpu.async_copy(itab.at[hiid_v.at[pl.ds(off + pos, ch)]],
                             hitem.at[pl.ds(sb + pos, ch)], sem_i)

    def wait_item(g, sb, sem_i):
        off = pl.multiple_of(g * ROWS_G, ROWS_G)
        for pos, ch in chunks:
            pltpu.make_async_copy(
                itab.at[hiid_v.at[pl.ds(off + pos, ch)]],
                hitem.at[pl.ds(sb + pos, ch)], sem_i).wait()

    def compute(g, sb, sem_i, sem_c):
        off = pl.multiple_of(g * ROWS_G, ROWS_G)
        cate_copies = []
        for pos, ch in chunks:
            cate_copies.append(pltpu.async_copy(
                ctab.at[haid_v.at[pl.ds(off + pos, ch)]],
                hcate.at[pl.ds(pos, ch)], sem_c))

        row3c = pv_v[3, pl.ds(0, 16)]
        c64s = row3c[0]
        dds = row3c[1]
        c0s = row3c[2]

        ge = g * 16 + lane
        qm = jnp.zeros((16,), _F32)
        qc = jnp.zeros((16,), _F32)
        nq2 = jnp.zeros((16,), _F32)
        r0 = jnp.zeros((16,), _I32)
        qiv = []
        for f in range(32):
            fx = jnp.bitwise_xor(lane, f)
            v = plsc.load_gather(qitem, [ge, fx])
            qiv.append(v)
            qm = qm + v * plsc.load_gather(pv_v, [r0, fx])
            qc = qc + v * plsc.load_gather(pv_v, [r0, 64 + fx])
            nq2 = nq2 + v * v

        wait_item(g, sb, sem_i)

        def t1_body(t, _):
            row = sb + lane50 + t
            A = jnp.zeros((16,), _F32)
            n2 = jnp.zeros((16,), _F32)
            for f in range(32):
                fx = jnp.bitwise_xor(lane, f)
                a = plsc.load_gather(hitem, [row, fx])
                A = A + a * qiv[f]
                n2 = n2 + a * a
            Abuf[t] = A
            Nbuf[t] = n2
            return 0

        lax.fori_loop(0, HIST, t1_body, 0, unroll=1)

        qcv = []
        for f in range(32):
            fx = jnp.bitwise_xor(lane, f)
            v = plsc.load_gather(qcate, [ge, fx])
            qcv.append(v)
            qm = qm + v * plsc.load_gather(pv_v, [r0, 32 + fx])
            qc = qc + v * plsc.load_gather(pv_v, [r0, 96 + fx])
            nq2 = nq2 + v * v
        qc = qc + dds

        for cp in cate_copies:
            cp.wait()

        def t2_body(t, _):
            row = lane50 + t
            A = Abuf[t]
            n2 = Nbuf[t]
            for f in range(32):
                fx = jnp.bitwise_xor(lane, f)
                b = plsc.load_gather(hcate, [row, fx])
                A = A + b * qcv[f]
                n2 = n2 + b * b
            Abuf[t] = A
            s = jnp.maximum(nq2 * n2, 1e-30)
            si = plsc.bitcast(s, _I32)
            y = plsc.bitcast(jnp.int32(0x5F3759DF) - (si >> 1), _F32)
            hs = 0.5 * s
            for _ in range(2):
                y = y * (1.5 - hs * y * y)
            simbuf[t] = A * y
            return 0

        lax.fori_loop(0, HIST, t2_body, 0, unroll=1)

        ssum = jnp.full((16,), 1e-8, _F32)
        for k in range(5):
            bv = neg
            bi = jnp.zeros((16,), _I32)
            for t in range(HIST):
                v = simbuf[t]
                better = v > bv
                bv = jnp.where(better, v, bv)
                bi = jnp.where(better, t, bi)
            plsc.store_scatter(simbuf, [bi, lane], neg)
            vbuf[k] = bv
            ibuf[k] = bi
            ssum = ssum + bv

        def k_body(k, contrib):
            bv = vbuf[k]
            bi = ibuf[k]
            wk = bv / ssum
            rowi = sb + lane50 + bi
            rowc = lane50 + bi
            gk = jnp.zeros((16,), _F32)
            pk = jnp.zeros((16,), _F32)
            r1 = jnp.full((16,), 1, _I32)
            for f in range(32):
                fx = jnp.bitwise_xor(lane, f)
                a = plsc.load_gather(hitem, [rowi, fx])
                b = plsc.load_gather(hcate, [rowc, fx])
                gk = (gk + a * plsc.load_gather(pv_v, [r1, fx])
                      + b * plsc.load_gather(pv_v, [r1, 32 + fx]))
                pk = (pk + a * plsc.load_gather(pv_v, [r1, 64 + fx])
                      + b * plsc.load_gather(pv_v, [r1, 96 + fx]))
            Ak = plsc.load_gather(Abuf, [bi, lane])
            rate = plsc.load_gather(hrate_v, [ge * HIST + bi])
            rk = plsc.load_gather(pv_v, [jnp.full((16,), 2, _I32), rate])
            au = wk * (gk + Ak * c64s) + qc
            return contrib + au * (wk * pk + rk)

        contrib = lax.fori_loop(0, 5, k_body, jnp.zeros((16,), _F32))
        zbuf[pl.ds(g * 16, 16)] = qm + contrib + c0s

    NJ = NG // 2

    def j_body(j, _):
        g0 = pl.multiple_of(2 * j, 2)
        compute(g0, 0, sem_i0, sem_c0)

        @pl.when(j < NJ - 1)
        def _():
            fire_item(g0 + 2, 0, sem_i0)

        compute(g0 + 1, ROWS_G, sem_i1, sem_c1)

        @pl.when(j < NJ - 1)
        def _():
            fire_item(g0 + 3, ROWS_G, sem_i1)

        return 0

    fire_item(0, 0, sem_i0)
    fire_item(1, ROWS_G, sem_i1)
    lax.fori_loop(0, NJ, j_body, 0, unroll=False)
    pltpu.sync_copy(zbuf, z_h.at[pl.ds(base, EPW)])


def _sc_call(itab, ctab, iid, aid, hiid, haid, hrate, pv, interpret=False):
    mesh = plsc.VectorSubcoreMesh(core_axis_name="c", subcore_axis_name="s",
                                  num_cores=2, num_subcores=16)
    f = pl.kernel(
        _sc_body,
        out_type=jax.ShapeDtypeStruct((B,), _F32),
        mesh=mesh,
        scratch_types=[
            pltpu.VMEM((EPW,), _I32),
            pltpu.VMEM((EPW,), _I32),
            pltpu.VMEM((EPW * HIST,), _I32),
            pltpu.VMEM((EPW * HIST,), _I32),
            pltpu.VMEM((EPW * HIST,), _I32),
            pltpu.VMEM((4, 128), _F32),
            pltpu.VMEM((EPW, H), _F32),
            pltpu.VMEM((EPW, H), _F32),
            pltpu.VMEM((2 * ROWS_G, H), _F32),
            pltpu.VMEM((ROWS_G, H), _F32),
            pltpu.VMEM((HIST, 16), _F32),
            pltpu.VMEM((HIST, 16), _F32),
            pltpu.VMEM((HIST, 16), _F32),
            pltpu.VMEM((8, 16), _F32),
            pltpu.VMEM((8, 16), _I32),
            pltpu.VMEM((EPW,), _F32),
            pltpu.SemaphoreType.DMA,
            pltpu.SemaphoreType.DMA,
            pltpu.SemaphoreType.DMA,
            pltpu.SemaphoreType.DMA,
        ],
        compiler_params=pltpu.CompilerParams(needs_layout_passes=False,
                                             use_tc_tiling_on_sc=False),
        interpret=interpret,
    )
    return f(itab, ctab, iid, aid, hiid, haid, hrate, pv)


def _final_body(z_ref, y_ref, probs_ref, loss_ref):
    z = z_ref[...]
    y = y_ref[...]
    probs_ref[...] = jax.nn.sigmoid(z)
    l = jnp.maximum(z, 0.0) - z * y + jnp.log1p(jnp.exp(-jnp.abs(z)))
    loss_ref[0, 0] = jnp.sum(l) * (1.0 / B)


def _final(z2d, y2d, interpret=False):
    return pl.pallas_call(
        _final_body,
        out_shape=(jax.ShapeDtypeStruct((B // 128, 128), _F32),
                   jax.ShapeDtypeStruct((1, 1), _F32)),
        out_specs=(pl.BlockSpec((B // 128, 128), lambda: (0, 0)),
                   pl.BlockSpec(memory_space=pltpu.SMEM)),
        interpret=interpret,
    )(z2d, y2d)


def _run(iid, aid, lb, hist_iid_seq, hist_aid_seq, hist_rate_seq,
         item_table, cate_table, rating_table,
         au_w1, au_b1, au_w2, au_b2,
         lin_w1, lin_b1, lin_w2, lin_b2, lin_w3, lin_b3,
         interpret=False):
    pv = _collapse((au_w2, au_b1.reshape(36, 1), au_b2.reshape(1, 1),
                    au_w1[:, 0:64], au_w1[:, 64:65], au_w1[:, 65:129],
                    lin_w1[:, 0:64], lin_w1[:, 64:128], lin_w1[:, 128:160],
                    lin_b1.reshape(80, 1), lin_w2, lin_b2.reshape(40, 1),
                    lin_w3, lin_b3.reshape(1, 1), rating_table),
                   interpret=interpret)
    z = _sc_call(item_table, cate_table,
                 iid.astype(_I32),
                 aid.reshape(B).astype(_I32),
                 hist_iid_seq.reshape(B * HIST).astype(_I32),
                 hist_aid_seq.reshape(B * HIST).astype(_I32),
                 hist_rate_seq.reshape(B * HIST).astype(_I32),
                 pv, interpret=interpret)
    y2d = lb.reshape(B // 128, 128).astype(_F32)
    probs2d, loss11 = _final(z.reshape(B // 128, 128), y2d,
                             interpret=interpret)
    return probs2d.reshape(B, 1), loss11[0, 0]


def kernel(iid, aid, lb, hist_iid_seq, hist_aid_seq, hist_rate_seq,
           item_table, cate_table, rating_table,
           au_w1, au_b1, au_w2, au_b2,
           lin_w1, lin_b1, lin_w2, lin_b2, lin_w3, lin_b3):
    return _run(iid, aid, lb, hist_iid_seq, hist_aid_seq, hist_rate_seq,
                item_table, cate_table, rating_table,
                au_w1, au_b1, au_w2, au_b2,
                lin_w1, lin_b1, lin_w2, lin_b2, lin_w3, lin_b3)


# speedup vs baseline: 1.2023x; 1.0424x over previous
import functools

import jax
import jax.numpy as jnp
from jax import lax
from jax.experimental import pallas as pl
from jax.experimental.pallas import tpu as pltpu
from jax.experimental.pallas import tpu_sc as plsc

B = 4096
H = 32
HIST = 50
NW = 32
EPW = B // NW
G = 16
NG = EPW // G
ROWS_G = G * HIST
_F32 = jnp.float32
_I32 = jnp.int32


def _collapse_body(au_w2, au_b1, au_b2, au_w1a, au_w1x, au_w1b,
                   lw1q, lw1h, lw1r, lb1, lw2, lb2, lw3, lb3, rtab,
                   out_ref):
    hi = jax.lax.Precision.HIGHEST
    dot = functools.partial(jnp.dot, precision=hi)
    v2 = 0.5 * au_w2[...]
    c_h = dot(v2, au_w1a[...])
    c64 = dot(v2, au_w1x[...])[0, 0]
    c_q = dot(v2, au_w1b[...])
    dd = dot(v2, au_b1[...])[0, 0] + au_b2[0, 0]
    w32 = 0.25 * dot(lw3[...], lw2[...])
    m_q = dot(w32, lw1q[...])
    m_h = dot(w32, lw1h[...])
    m_r = dot(w32, lw1r[...])
    c0 = (dot(w32, lb1[...])[0, 0]
          + 0.5 * dot(lw3[...], lb2[...])[0, 0] + lb3[0, 0])
    rho = lax.dot_general(m_r, rtab[...], (((1,), (1,)), ((), ())),
                          precision=hi)
    row0 = jnp.concatenate([m_q, c_q], axis=1)
    row1 = jnp.concatenate([c_h, m_h], axis=1)
    row2 = jnp.concatenate([rho, jnp.zeros((1, 118), _F32)], axis=1)
    i = lax.broadcasted_iota(_I32, (1, 128), 1)
    row3 = (jnp.where(i == 0, c64, 0.0) + jnp.where(i == 1, dd, 0.0)
            + jnp.where(i == 2, c0, 0.0))
    out_ref[...] = jnp.concatenate([row0, row1, row2, row3], axis=0)


def _collapse(d, interpret=False):
    return pl.pallas_call(
        _collapse_body,
        out_shape=jax.ShapeDtypeStruct((4, 128), _F32),
        interpret=interpret,
    )(*d)


def _sc_body(itab, ctab, iid_h, aid_h, hiid_h, haid_h, hrate_h, pv_h,
             z_h,
             iid_v, aid_v, hiid_v, haid_v, hrate_v, pv_v,
             qitem, qcate, hitem, hcate, Abuf, Nbuf, simbuf,
             vbuf, ibuf, zbuf, swb, sem_i0, sem_c0, sem_i1, sem_c1):
    cid = lax.axis_index("c")
    sid = lax.axis_index("s")
    wid = sid * 2 + cid
    base = pl.multiple_of(wid * EPW, EPW)
    hbase = pl.multiple_of(wid * (EPW * HIST), EPW * HIST)

    pltpu.sync_copy(pv_h, pv_v)
    pltpu.sync_copy(iid_h.at[pl.ds(base, EPW)], iid_v)
    pltpu.sync_copy(aid_h.at[pl.ds(base, EPW)], aid_v)
    pltpu.sync_copy(hiid_h.at[pl.ds(hbase, EPW * HIST)], hiid_v)
    pltpu.sync_copy(haid_h.at[pl.ds(hbase, EPW * HIST)], haid_v)
    pltpu.sync_copy(hrate_h.at[pl.ds(hbase, EPW * HIST)], hrate_v)
    pltpu.async_copy(itab.at[iid_v], qitem, sem_i0).wait()
    pltpu.async_copy(ctab.at[aid_v], qcate, sem_c0).wait()

    lane = lax.iota(_I32, 16)
    lane50 = lane * HIST
    neg = jnp.full((16,), -3.0e38, _F32)

    for blk in range(8):
        rv = jnp.full((16,), 0 if blk < 4 else 1, _I32)
        seg = (blk % 4) * 32
        for f in range(32):
            fx = jnp.bitwise_xor(lane, f)
            swb[blk * 32 + f] = plsc.load_gather(pv_v, [rv, seg + fx])
    chunks = ((0, 128), (128, 128), (256, 128), (384, 128),
              (512, 128), (640, 128), (768, 32))

    def fire_item(g, sb, sem_i):
        off = pl.multiple_of(g * ROWS_G, ROWS_G)
        for pos, ch in chunks:
            pltpu.async_copy(itab.at[hiid_v.at[pl.ds(off + pos, ch)]],
                             hitem.at[pl.ds(sb + pos, ch)], sem_i)

    def wait_item(g, sb, sem_i):
        off = pl.multiple_of(g * ROWS_G, ROWS_G)
        for pos, ch in chunks:
            pltpu.make_async_copy(
                itab.at[hiid_v.at[pl.ds(off + pos, ch)]],
                hitem.at[pl.ds(sb + pos, ch)], sem_i).wait()

    def compute(g, sb, sem_i, sem_c):
        off = pl.multiple_of(g * ROWS_G, ROWS_G)
        cate_copies = []
        for pos, ch in chunks:
            cate_copies.append(pltpu.async_copy(
                ctab.at[haid_v.at[pl.ds(off + pos, ch)]],
                hcate.at[pl.ds(pos, ch)], sem_c))

        row3c = pv_v[3, pl.ds(0, 16)]
        c64s = row3c[0]
        dds = row3c[1]
        c0s = row3c[2]

        ge = g * 16 + lane
        qm = jnp.zeros((16,), _F32)
        qc = jnp.zeros((16,), _F32)
        nq2 = jnp.zeros((16,), _F32)
        qiv = []
        for f in range(32):
            fx = jnp.bitwise_xor(lane, f)
            v = plsc.load_gather(qitem, [ge, fx])
            qiv.append(v)
            qm = qm + v * swb[f]
            qc = qc + v * swb[64 + f]
            nq2 = nq2 + v * v

        wait_item(g, sb, sem_i)

        def t1_body(t, _):
            row = sb + lane50 + t
            A = jnp.zeros((16,), _F32)
            n2 = jnp.zeros((16,), _F32)
            for f in range(32):
                fx = jnp.bitwise_xor(lane, f)
                a = plsc.load_gather(hitem, [row, fx])
                A = A + a * qiv[f]
                n2 = n2 + a * a
            Abuf[t] = A
            Nbuf[t] = n2
            return 0

        lax.fori_loop(0, HIST, t1_body, 0, unroll=1)

        qcv = []
        for f in range(32):
            fx = jnp.bitwise_xor(lane, f)
            v = plsc.load_gather(qcate, [ge, fx])
            qcv.append(v)
            qm = qm + v * swb[32 + f]
            qc = qc + v * swb[96 + f]
            nq2 = nq2 + v * v
        qc = qc + dds

        for cp in cate_copies:
            cp.wait()

        def t2_body(t, _):
            row = lane50 + t
            A = Abuf[t]
            n2 = Nbuf[t]
            for f in range(32):
                fx = jnp.bitwise_xor(lane, f)
                b = plsc.load_gather(hcate, [row, fx])
                A = A + b * qcv[f]
                n2 = n2 + b * b
            Abuf[t] = A
            s = jnp.maximum(nq2 * n2, 1e-30)
            si = plsc.bitcast(s, _I32)
            y = plsc.bitcast(jnp.int32(0x5F3759DF) - (si >> 1), _F32)
            hs = 0.5 * s
            for _ in range(2):
                y = y * (1.5 - hs * y * y)
            simbuf[t] = A * y
            return 0

        lax.fori_loop(0, HIST, t2_body, 0, unroll=1)

        ssum = jnp.full((16,), 1e-8, _F32)
        for k in range(5):
            bv = neg
            bi = jnp.zeros((16,), _I32)
            for t in range(HIST):
                v = simbuf[t]
                better = v > bv
                bv = jnp.where(better, v, bv)
                bi = jnp.where(better, t, bi)
            plsc.store_scatter(simbuf, [bi, lane], neg)
            vbuf[k] = bv
            ibuf[k] = bi
            ssum = ssum + bv

        def k_body(k, contrib):
            bv = vbuf[k]
            bi = ibuf[k]
            wk = bv / ssum
            rowi = sb + lane50 + bi
            rowc = lane50 + bi
            gk = jnp.zeros((16,), _F32)
            pk = jnp.zeros((16,), _F32)
            for f in range(32):
                fx = jnp.bitwise_xor(lane, f)
                a = plsc.load_gather(hitem, [rowi, fx])
                b = plsc.load_gather(hcate, [rowc, fx])
                gk = gk + a * swb[128 + f] + b * swb[160 + f]
                pk = pk + a * swb[192 + f] + b * swb[224 + f]
            Ak = plsc.load_gather(Abuf, [bi, lane])
            rate = plsc.load_gather(hrate_v, [ge * HIST + bi])
            rk = plsc.load_gather(pv_v, [jnp.full((16,), 2, _I32), rate])
            au = wk * (gk + Ak * c64s) + qc
            return contrib + au * (wk * pk + rk)

        contrib = lax.fori_loop(0, 5, k_body, jnp.zeros((16,), _F32))
        zbuf[pl.ds(g * 16, 16)] = qm + contrib + c0s

    NJ = NG // 2

    def j_body(j, _):
        g0 = pl.multiple_of(2 * j, 2)
        compute(g0, 0, sem_i0, sem_c0)

        @pl.when(j < NJ - 1)
        def _():
            fire_item(g0 + 2, 0, sem_i0)

        compute(g0 + 1, ROWS_G, sem_i1, sem_c1)

        @pl.when(j < NJ - 1)
        def _():
            fire_item(g0 + 3, ROWS_G, sem_i1)

        return 0

    fire_item(0, 0, sem_i0)
    fire_item(1, ROWS_G, sem_i1)
    lax.fori_loop(0, NJ, j_body, 0, unroll=False)
    pltpu.sync_copy(zbuf, z_h.at[pl.ds(base, EPW)])


def _sc_call(itab, ctab, iid, aid, hiid, haid, hrate, pv, interpret=False):
    mesh = plsc.VectorSubcoreMesh(core_axis_name="c", subcore_axis_name="s",
                                  num_cores=2, num_subcores=16)
    f = pl.kernel(
        _sc_body,
        out_type=jax.ShapeDtypeStruct((B,), _F32),
        mesh=mesh,
        scratch_types=[
            pltpu.VMEM((EPW,), _I32),
            pltpu.VMEM((EPW,), _I32),
            pltpu.VMEM((EPW * HIST,), _I32),
            pltpu.VMEM((EPW * HIST,), _I32),
            pltpu.VMEM((EPW * HIST,), _I32),
            pltpu.VMEM((4, 128), _F32),
            pltpu.VMEM((EPW, H), _F32),
            pltpu.VMEM((EPW, H), _F32),
            pltpu.VMEM((2 * ROWS_G, H), _F32),
            pltpu.VMEM((ROWS_G, H), _F32),
            pltpu.VMEM((HIST, 16), _F32),
            pltpu.VMEM((HIST, 16), _F32),
            pltpu.VMEM((HIST, 16), _F32),
            pltpu.VMEM((8, 16), _F32),
            pltpu.VMEM((8, 16), _I32),
            pltpu.VMEM((EPW,), _F32),
            pltpu.VMEM((256, 16), _F32),
            pltpu.SemaphoreType.DMA,
            pltpu.SemaphoreType.DMA,
            pltpu.SemaphoreType.DMA,
            pltpu.SemaphoreType.DMA,
        ],
        compiler_params=pltpu.CompilerParams(needs_layout_passes=False,
                                             use_tc_tiling_on_sc=False),
        interpret=interpret,
    )
    return f(itab, ctab, iid, aid, hiid, haid, hrate, pv)


def _final_body(z_ref, y_ref, probs_ref, loss_ref):
    z = z_ref[...]
    y = y_ref[...]
    probs_ref[...] = jax.nn.sigmoid(z)
    l = jnp.maximum(z, 0.0) - z * y + jnp.log1p(jnp.exp(-jnp.abs(z)))
    loss_ref[0, 0] = jnp.sum(l) * (1.0 / B)


def _final(z2d, y2d, interpret=False):
    return pl.pallas_call(
        _final_body,
        out_shape=(jax.ShapeDtypeStruct((B // 128, 128), _F32),
                   jax.ShapeDtypeStruct((1, 1), _F32)),
        out_specs=(pl.BlockSpec((B // 128, 128), lambda: (0, 0)),
                   pl.BlockSpec(memory_space=pltpu.SMEM)),
        interpret=interpret,
    )(z2d, y2d)


def _run(iid, aid, lb, hist_iid_seq, hist_aid_seq, hist_rate_seq,
         item_table, cate_table, rating_table,
         au_w1, au_b1, au_w2, au_b2,
         lin_w1, lin_b1, lin_w2, lin_b2, lin_w3, lin_b3,
         interpret=False):
    pv = _collapse((au_w2, au_b1.reshape(36, 1), au_b2.reshape(1, 1),
                    au_w1[:, 0:64], au_w1[:, 64:65], au_w1[:, 65:129],
                    lin_w1[:, 0:64], lin_w1[:, 64:128], lin_w1[:, 128:160],
                    lin_b1.reshape(80, 1), lin_w2, lin_b2.reshape(40, 1),
                    lin_w3, lin_b3.reshape(1, 1), rating_table),
                   interpret=interpret)
    z = _sc_call(item_table, cate_table,
                 iid.astype(_I32),
                 aid.reshape(B).astype(_I32),
                 hist_iid_seq.reshape(B * HIST).astype(_I32),
                 hist_aid_seq.reshape(B * HIST).astype(_I32),
                 hist_rate_seq.reshape(B * HIST).astype(_I32),
                 pv, interpret=interpret)
    y2d = lb.reshape(B // 128, 128).astype(_F32)
    probs2d, loss11 = _final(z.reshape(B // 128, 128), y2d,
                             interpret=interpret)
    return probs2d.reshape(B, 1), loss11[0, 0]


def kernel(iid, aid, lb, hist_iid_seq, hist_aid_seq, hist_rate_seq,
           item_table, cate_table, rating_table,
           au_w1, au_b1, au_w2, au_b2,
           lin_w1, lin_b1, lin_w2, lin_b2, lin_w3, lin_b3):
    return _run(iid, aid, lb, hist_iid_seq, hist_aid_seq, hist_rate_seq,
                item_table, cate_table, rating_table,
                au_w1, au_b1, au_w2, au_b2,
                lin_w1, lin_b1, lin_w2, lin_b2, lin_w3, lin_b3)
